# Initial kernel scaffold; baseline (speedup 1.0000x reference)
import jax
import jax.numpy as jnp
from jax.experimental import pallas as pl


def kernel(x, Wq1, bq1, Wq2, bq2, subkey_one, subkey_two, values, W1, W2):
    raise NotImplementedError("write your pallas kernel here")


# speedup vs baseline: 2.9171x; 2.9171x over previous
import functools

import jax
import jax.numpy as jnp
from jax import lax
from jax.experimental import pallas as pl
from jax.experimental.pallas import tpu as pltpu
from jax.experimental.pallas import tpu_sc as plsc

F32 = jnp.float32
I32 = jnp.int32

D = 1024
HID = 4096
SKD = 256
NSK = 256
VD = 1024
TK = 32

TOKB = 256
HIDB = 1024
NHB = HID // HIDB
CW = TK * TK

NC, NS = 2, 16
NW = NC * NS


def _rms(v, axis=-1):
    return v * lax.rsqrt(jnp.mean(v * v, axis=axis, keepdims=True) + 1e-6)


def _silu(v):
    return v * (1.0 / (1.0 + jnp.exp(-v)))


def _extract_topk(s_scr, idx_ref, c_scr, sel, width, k):
    iota_w = lax.broadcasted_iota(I32, s_scr.shape, 1)
    iota_k = lax.broadcasted_iota(I32, idx_ref.shape, 1)

    def body(m, _):
        s = s_scr[...]
        mx = jnp.max(s, axis=1, keepdims=True)
        am = jnp.min(jnp.where(s == mx, iota_w, width), axis=1, keepdims=True)
        idx_ref[...] = jnp.where(iota_k == m, am, idx_ref[...])
        c_scr[...] += jnp.where(sel == m, mx, 0.0)
        s_scr[...] = jnp.where(iota_w == am, -jnp.inf, s)
        return 0

    lax.fori_loop(0, k, body, 0)


def _mlp_topk_body(x_ref, wq1_ref, bq1_ref, wq2_ref, bq2_ref, sk1_ref, sk2_ref,
                   fidx_ref, w_ref,
                   qacc, s_scr, c_scr, i1_scr, i2_scr):
    hb = pl.program_id(1)

    @pl.when(hb == 0)
    def _init():
        qacc[...] = jnp.zeros_like(qacc)

    h = _silu(jnp.dot(x_ref[...], wq1_ref[...], preferred_element_type=F32)
              + bq1_ref[...])
    qacc[...] += jnp.dot(h, wq2_ref[...], preferred_element_type=F32)

    @pl.when(hb == NHB - 1)
    def _finish():
        q = _rms(qacc[...] + bq2_ref[...])
        k1 = _rms(sk1_ref[...])
        k2 = _rms(sk2_ref[...])
        dn = (((1,), (1,)), ((), ()))
        iota_cw = lax.broadcasted_iota(I32, (TOKB, CW), 1)
        c_scr[...] = jnp.zeros((TOKB, CW), F32)
        s_scr[...] = lax.dot_general(q, k1, dn, preferred_element_type=F32)
        _extract_topk(s_scr, i1_scr, c_scr, iota_cw // TK, NSK, TK)
        s_scr[...] = lax.dot_general(q, k2, dn, preferred_element_type=F32)
        _extract_topk(s_scr, i2_scr, c_scr, iota_cw % TK, NSK, TK)

        iota_k = lax.broadcasted_iota(I32, (TOKB, TK), 1)
        i1 = i1_scr[...]
        i2 = i2_scr[...]

        def body(m, _):
            c = c_scr[...]
            mx = jnp.max(c, axis=1, keepdims=True)
            am = jnp.min(jnp.where(c == mx, iota_cw, CW), axis=1, keepdims=True)
            c_scr[...] = jnp.where(iota_cw == am, -jnp.inf, c)
            row = am // TK
            col = am % TK
            idx1 = jnp.sum(jnp.where(iota_k == row, i1, 0), axis=1, keepdims=True)
            idx2 = jnp.sum(jnp.where(iota_k == col, i2, 0), axis=1, keepdims=True)
            fidx_ref[...] = jnp.where(iota_k == m, idx1 * NSK + idx2, fidx_ref[...])
            w_ref[...] = jnp.where(iota_k == m, mx, w_ref[...])
            return 0

        lax.fori_loop(0, TK, body, 0)

        cs = w_ref[...]
        e = jnp.exp(cs - jnp.max(cs, axis=1, keepdims=True))
        w_ref[...] = e / jnp.sum(e, axis=1, keepdims=True)


def _mlp_topk(x2, wq1, bq1, wq2, bq2, sk1, sk2, interpret=False):
    n = x2.shape[0]
    ntb = n // TOKB
    grid = (ntb, NHB)
    return pl.pallas_call(
        _mlp_topk_body,
        grid=grid,
        in_specs=[
            pl.BlockSpec((TOKB, D), lambda tb, hb: (tb, 0)),
            pl.BlockSpec((D, HIDB), lambda tb, hb: (0, hb)),
            pl.BlockSpec((1, HIDB), lambda tb, hb: (0, hb)),
            pl.BlockSpec((HIDB, SKD), lambda tb, hb: (hb, 0)),
            pl.BlockSpec((1, SKD), lambda tb, hb: (0, 0)),
            pl.BlockSpec((NSK, SKD), lambda tb, hb: (0, 0)),
            pl.BlockSpec((NSK, SKD), lambda tb, hb: (0, 0)),
        ],
        out_specs=[
            pl.BlockSpec((TOKB, TK), lambda tb, hb: (tb, 0)),
            pl.BlockSpec((TOKB, TK), lambda tb, hb: (tb, 0)),
        ],
        out_shape=[
            jax.ShapeDtypeStruct((n, TK), I32),
            jax.ShapeDtypeStruct((n, TK), F32),
        ],
        scratch_shapes=[
            pltpu.VMEM((TOKB, SKD), F32),
            pltpu.VMEM((TOKB, NSK), F32),
            pltpu.VMEM((TOKB, CW), F32),
            pltpu.VMEM((TOKB, TK), I32),
            pltpu.VMEM((TOKB, TK), I32),
        ],
        compiler_params=pltpu.CompilerParams(
            dimension_semantics=("parallel", "arbitrary")),
        interpret=interpret,
    )(x2, wq1, bq1, wq2, bq2, sk1, sk2)


def _out_mlp_body(x_ref, y_ref, w1_ref, w2_ref, o_ref):
    m = _silu(jnp.dot(x_ref[...], w1_ref[...], preferred_element_type=F32))
    o_ref[...] = jnp.dot(y_ref[...] * m, w2_ref[...], preferred_element_type=F32)


def _out_mlp(x2, y, w1, w2, interpret=False):
    n = x2.shape[0]
    return pl.pallas_call(
        _out_mlp_body,
        grid=(n // TOKB,),
        in_specs=[
            pl.BlockSpec((TOKB, D), lambda tb: (tb, 0)),
            pl.BlockSpec((TOKB, VD), lambda tb: (tb, 0)),
            pl.BlockSpec((D, VD), lambda tb: (0, 0)),
            pl.BlockSpec((VD, D), lambda tb: (0, 0)),
        ],
        out_specs=pl.BlockSpec((TOKB, D), lambda tb: (tb, 0)),
        out_shape=jax.ShapeDtypeStruct((n, D), F32),
        compiler_params=pltpu.CompilerParams(
            dimension_semantics=("parallel",)),
        interpret=interpret,
    )(x2, y, w1, w2)


def _sc_weighted_gather(values, fidx, w16):
    n = fidx.shape[0]
    tpw = n // NW
    mesh = plsc.VectorSubcoreMesh(core_axis_name="c", subcore_axis_name="s")

    @functools.partial(
        pl.kernel, mesh=mesh,
        out_type=jax.ShapeDtypeStruct((n, VD), F32),
        scratch_types=[
            pltpu.VMEM((tpw, TK), I32),
            pltpu.VMEM((TK, VD), F32),
            pltpu.VMEM((TK, 16), F32),
            pltpu.VMEM((VD,), F32),
            pltpu.SemaphoreType.DMA,
            pltpu.SemaphoreType.DMA,
        ],
    )
    def k(values_hbm, fidx_hbm, w_hbm, out_hbm, idx_v, rows_v, w_v, y_v,
          sem_g, sem_w):
        wid = lax.axis_index("s") * NC + lax.axis_index("c")
        base = wid * tpw
        pltpu.sync_copy(fidx_hbm.at[pl.ds(base, tpw)], idx_v)

        def tok(t, _):
            cp_g = pltpu.async_copy(values_hbm.at[idx_v.at[t]], rows_v, sem_g)
            cp_w = pltpu.async_copy(w_hbm.at[base + t], w_v, sem_w)
            cp_w.wait()
            cp_g.wait()
            for dblk in range(4):
                def kbody(kk, accs):
                    wk = w_v[kk, :]
                    return tuple(
                        accs[d] + rows_v[kk, pl.ds(dblk * 256 + d * 16, 16)] * wk
                        for d in range(16))
                accs = lax.fori_loop(
                    0, TK, kbody,
                    tuple(jnp.zeros((16,), F32) for _ in range(16)))
                for d in range(16):
                    y_v[pl.ds(dblk * 256 + d * 16, 16)] = accs[d]
            pltpu.sync_copy(y_v, out_hbm.at[base + t])
            return 0

        lax.fori_loop(0, tpw, tok, 0)

    return k(values, fidx, w16)


def kernel(x, Wq1, bq1, Wq2, bq2, subkey_one, subkey_two, values, W1, W2):
    b, s, d = x.shape
    n = b * s
    x2 = x.reshape(n, d)
    fidx, w = _mlp_topk(x2, Wq1, bq1.reshape(1, -1), Wq2, bq2.reshape(1, -1),
                        subkey_one, subkey_two)
    w16 = jnp.broadcast_to(w[:, :, None], (n, TK, 16))
    y = _sc_weighted_gather(values, fidx, w16)
    out2 = _out_mlp(x2, y, W1, W2)
    return out2.reshape(b, s, d)


# speedup vs baseline: 3.4772x; 1.1920x over previous
import functools

import jax
import jax.numpy as jnp
from jax import lax
from jax.experimental import pallas as pl
from jax.experimental.pallas import tpu as pltpu
from jax.experimental.pallas import tpu_sc as plsc

F32 = jnp.float32
I32 = jnp.int32

D = 1024
HID = 4096
SKD = 256
NSK = 256
VD = 1024
TK = 32

TOKB = 256
HIDB = 1024
NHB = HID // HIDB
CW = TK * TK

NC, NS = 2, 16
NW = NC * NS


def _rms(v, axis=-1):
    return v * lax.rsqrt(jnp.mean(v * v, axis=axis, keepdims=True) + 1e-6)


def _silu(v):
    return v * (1.0 / (1.0 + jnp.exp(-v)))


def _extract_topk(s_scr, idx_ref, c_scr, sel, width, k):
    iota_w = lax.broadcasted_iota(I32, s_scr.shape, 1)
    iota_k = lax.broadcasted_iota(I32, idx_ref.shape, 1)

    def body(m, _):
        s = s_scr[...]
        mx = jnp.max(s, axis=1, keepdims=True)
        am = jnp.min(jnp.where(s == mx, iota_w, width), axis=1, keepdims=True)
        idx_ref[...] = jnp.where(iota_k == m, am, idx_ref[...])
        c_scr[...] += jnp.where(sel == m, mx, 0.0)
        s_scr[...] = jnp.where(iota_w == am, -jnp.inf, s)
        return 0

    lax.fori_loop(0, k, body, 0)


def _mlp_topk_body(x_ref, wq1_ref, bq1_ref, wq2_ref, bq2_ref, sk1_ref, sk2_ref,
                   fidx_ref, w_ref,
                   qacc, s_scr, c_scr, i1_scr, i2_scr):
    hb = pl.program_id(1)

    @pl.when(hb == 0)
    def _init():
        qacc[...] = jnp.zeros_like(qacc)

    h = _silu(jnp.dot(x_ref[...], wq1_ref[...], preferred_element_type=F32)
              + bq1_ref[...])
    qacc[...] += jnp.dot(h, wq2_ref[...], preferred_element_type=F32)

    @pl.when(hb == NHB - 1)
    def _finish():
        q = _rms(qacc[...] + bq2_ref[...])
        k1 = _rms(sk1_ref[...])
        k2 = _rms(sk2_ref[...])
        dn = (((1,), (1,)), ((), ()))
        iota_cw = lax.broadcasted_iota(I32, (TOKB, CW), 1)
        c_scr[...] = jnp.zeros((TOKB, CW), F32)
        s_scr[...] = lax.dot_general(q, k1, dn, preferred_element_type=F32)
        _extract_topk(s_scr, i1_scr, c_scr, iota_cw // TK, NSK, TK)
        s_scr[...] = lax.dot_general(q, k2, dn, preferred_element_type=F32)
        _extract_topk(s_scr, i2_scr, c_scr, iota_cw % TK, NSK, TK)

        iota_k = lax.broadcasted_iota(I32, (TOKB, TK), 1)
        i1 = i1_scr[...]
        i2 = i2_scr[...]

        def body(m, _):
            c = c_scr[...]
            mx = jnp.max(c, axis=1, keepdims=True)
            am = jnp.min(jnp.where(c == mx, iota_cw, CW), axis=1, keepdims=True)
            c_scr[...] = jnp.where(iota_cw == am, -jnp.inf, c)
            row = am // TK
            col = am % TK
            idx1 = jnp.sum(jnp.where(iota_k == row, i1, 0), axis=1, keepdims=True)
            idx2 = jnp.sum(jnp.where(iota_k == col, i2, 0), axis=1, keepdims=True)
            fidx_ref[...] = jnp.where(iota_k == m, idx1 * NSK + idx2, fidx_ref[...])
            w_ref[...] = jnp.where(iota_k == m, mx, w_ref[...])
            return 0

        lax.fori_loop(0, TK, body, 0)

        cs = w_ref[...]
        e = jnp.exp(cs - jnp.max(cs, axis=1, keepdims=True))
        w_ref[...] = e / jnp.sum(e, axis=1, keepdims=True)


def _mlp_topk(x2, wq1, bq1, wq2, bq2, sk1, sk2, interpret=False):
    n = x2.shape[0]
    ntb = n // TOKB
    grid = (ntb, NHB)
    return pl.pallas_call(
        _mlp_topk_body,
        grid=grid,
        in_specs=[
            pl.BlockSpec((TOKB, D), lambda tb, hb: (tb, 0)),
            pl.BlockSpec((D, HIDB), lambda tb, hb: (0, hb)),
            pl.BlockSpec((1, HIDB), lambda tb, hb: (0, hb)),
            pl.BlockSpec((HIDB, SKD), lambda tb, hb: (hb, 0)),
            pl.BlockSpec((1, SKD), lambda tb, hb: (0, 0)),
            pl.BlockSpec((NSK, SKD), lambda tb, hb: (0, 0)),
            pl.BlockSpec((NSK, SKD), lambda tb, hb: (0, 0)),
        ],
        out_specs=[
            pl.BlockSpec((TOKB, TK), lambda tb, hb: (tb, 0)),
            pl.BlockSpec((TOKB, TK), lambda tb, hb: (tb, 0)),
        ],
        out_shape=[
            jax.ShapeDtypeStruct((n, TK), I32),
            jax.ShapeDtypeStruct((n, TK), F32),
        ],
        scratch_shapes=[
            pltpu.VMEM((TOKB, SKD), F32),
            pltpu.VMEM((TOKB, NSK), F32),
            pltpu.VMEM((TOKB, CW), F32),
            pltpu.VMEM((TOKB, TK), I32),
            pltpu.VMEM((TOKB, TK), I32),
        ],
        compiler_params=pltpu.CompilerParams(
            dimension_semantics=("parallel", "arbitrary")),
        interpret=interpret,
    )(x2, wq1, bq1, wq2, bq2, sk1, sk2)


def _out_mlp_body(x_ref, y_ref, w1_ref, w2_ref, o_ref):
    m = _silu(jnp.dot(x_ref[...], w1_ref[...], preferred_element_type=F32))
    o_ref[...] = jnp.dot(y_ref[...] * m, w2_ref[...], preferred_element_type=F32)


def _out_mlp(x2, y, w1, w2, interpret=False):
    n = x2.shape[0]
    return pl.pallas_call(
        _out_mlp_body,
        grid=(n // TOKB,),
        in_specs=[
            pl.BlockSpec((TOKB, D), lambda tb: (tb, 0)),
            pl.BlockSpec((TOKB, VD), lambda tb: (tb, 0)),
            pl.BlockSpec((D, VD), lambda tb: (0, 0)),
            pl.BlockSpec((VD, D), lambda tb: (0, 0)),
        ],
        out_specs=pl.BlockSpec((TOKB, D), lambda tb: (tb, 0)),
        out_shape=jax.ShapeDtypeStruct((n, D), F32),
        compiler_params=pltpu.CompilerParams(
            dimension_semantics=("parallel",)),
        interpret=interpret,
    )(x2, y, w1, w2)


def _sc_weighted_gather(values, fidx, w16):
    n = fidx.shape[0]
    tpw = n // NW
    mesh = plsc.VectorSubcoreMesh(core_axis_name="c", subcore_axis_name="s")

    @functools.partial(
        pl.kernel, mesh=mesh,
        out_type=jax.ShapeDtypeStruct((n, VD), F32),
        scratch_types=[
            pltpu.VMEM((tpw, TK), I32),
            pltpu.VMEM((TK, VD), F32),
            pltpu.VMEM((TK, VD), F32),
            pltpu.VMEM((TK, 16), F32),
            pltpu.VMEM((TK, 16), F32),
            pltpu.VMEM((VD,), F32),
            pltpu.VMEM((VD,), F32),
            pltpu.SemaphoreType.DMA,
            pltpu.SemaphoreType.DMA,
            pltpu.SemaphoreType.DMA,
            pltpu.SemaphoreType.DMA,
            pltpu.SemaphoreType.DMA,
            pltpu.SemaphoreType.DMA,
        ],
    )
    def k(values_hbm, fidx_hbm, w_hbm, out_hbm, idx_v,
          rows_a, rows_b, w_a, w_b, y_a, y_b,
          sga, sgb, swa, swb, sya, syb):
        wid = lax.axis_index("s") * NC + lax.axis_index("c")
        base = wid * tpw
        pltpu.sync_copy(fidx_hbm.at[pl.ds(base, tpw)], idx_v)

        def start_fetch(t, rows_v, w_v, sg, sw):
            pltpu.make_async_copy(values_hbm.at[idx_v.at[t]], rows_v, sg).start()
            pltpu.make_async_copy(w_hbm.at[base + t], w_v, sw).start()

        def wait_fetch(t, rows_v, w_v, sg, sw):
            pltpu.make_async_copy(values_hbm.at[idx_v.at[t]], rows_v, sg).wait()
            pltpu.make_async_copy(w_hbm.at[base + t], w_v, sw).wait()

        def compute(rows_v, w_v, y_v):
            for dblk in range(4):
                def kbody(kk, accs):
                    wk = w_v[kk, :]
                    return tuple(
                        accs[d] + rows_v[kk, pl.ds(dblk * 256 + d * 16, 16)] * wk
                        for d in range(16))
                accs = lax.fori_loop(
                    0, TK, kbody,
                    tuple(jnp.zeros((16,), F32) for _ in range(16)))
                for d in range(16):
                    y_v[pl.ds(dblk * 256 + d * 16, 16)] = accs[d]

        start_fetch(0, rows_a, w_a, sga, swa)
        start_fetch(1, rows_b, w_b, sgb, swb)

        def pair(g, _):
            for (t, rows_v, w_v, y_v, sg, sw, sy) in (
                    (2 * g, rows_a, w_a, y_a, sga, swa, sya),
                    (2 * g + 1, rows_b, w_b, y_b, sgb, swb, syb)):
                wait_fetch(t, rows_v, w_v, sg, sw)

                @pl.when(g > 0)
                def _drain():
                    pltpu.make_async_copy(y_v, out_hbm.at[base + t], sy).wait()

                compute(rows_v, w_v, y_v)
                pltpu.make_async_copy(y_v, out_hbm.at[base + t], sy).start()

                @pl.when(t + 2 < tpw)
                def _next():
                    start_fetch(t + 2, rows_v, w_v, sg, sw)
            return 0

        lax.fori_loop(0, tpw // 2, pair, 0)
        pltpu.make_async_copy(y_a, out_hbm.at[base], sya).wait()
        pltpu.make_async_copy(y_b, out_hbm.at[base], syb).wait()

    return k(values, fidx, w16)


def kernel(x, Wq1, bq1, Wq2, bq2, subkey_one, subkey_two, values, W1, W2):
    b, s, d = x.shape
    n = b * s
    x2 = x.reshape(n, d)
    fidx, w = _mlp_topk(x2, Wq1, bq1.reshape(1, -1), Wq2, bq2.reshape(1, -1),
                        subkey_one, subkey_two)
    w16 = jnp.broadcast_to(w[:, :, None], (n, TK, 16))
    y = _sc_weighted_gather(values, fidx, w16)
    out2 = _out_mlp(x2, y, W1, W2)
    return out2.reshape(b, s, d)


# speedup vs baseline: 3.6123x; 1.0388x over previous
import functools

import jax
import jax.numpy as jnp
from jax import lax
from jax.experimental import pallas as pl
from jax.experimental.pallas import tpu as pltpu
from jax.experimental.pallas import tpu_sc as plsc

F32 = jnp.float32
I32 = jnp.int32

D = 1024
HID = 4096
SKD = 256
NSK = 256
VD = 1024
TK = 32

TOKB = 256
HIDB = 1024
NHB = HID // HIDB
CW = 128 + 8 * TK

NC, NS = 2, 16
NW = NC * NS


def _rms(v, axis=-1):
    return v * lax.rsqrt(jnp.mean(v * v, axis=axis, keepdims=True) + 1e-6)


def _silu(v):
    return v * (1.0 / (1.0 + jnp.exp(-v)))


def _extract_topk(s_scr, vals_ref, idx_ref, width, k):
    iota_w = lax.broadcasted_iota(I32, s_scr.shape, 1)
    iota_k = lax.broadcasted_iota(I32, idx_ref.shape, 1)

    def body(m, _):
        s = s_scr[...]
        mx = jnp.max(s, axis=1, keepdims=True)
        am = jnp.min(jnp.where(s == mx, iota_w, width), axis=1, keepdims=True)
        vals_ref[...] = jnp.where(iota_k == m, mx, vals_ref[...])
        idx_ref[...] = jnp.where(iota_k == m, am, idx_ref[...])
        s_scr[...] = jnp.where(iota_w == am, -jnp.inf, s)
        return 0

    lax.fori_loop(0, k, body, 0)


def _mlp_topk_body(x_ref, wq1_ref, bq1_ref, wq2_ref, bq2_ref, sk1_ref, sk2_ref,
                   fidx_ref, w_ref,
                   qacc, s_scr, c_scr, v1_scr, i1_scr, v2_scr, i2_scr):
    hb = pl.program_id(1)

    @pl.when(hb == 0)
    def _init():
        qacc[...] = jnp.zeros_like(qacc)

    h = _silu(jnp.dot(x_ref[...], wq1_ref[...], preferred_element_type=F32)
              + bq1_ref[...])
    qacc[...] += jnp.dot(h, wq2_ref[...], preferred_element_type=F32)

    @pl.when(hb == NHB - 1)
    def _finish():
        q = _rms(qacc[...] + bq2_ref[...])
        k1 = _rms(sk1_ref[...])
        k2 = _rms(sk2_ref[...])
        dn = (((1,), (1,)), ((), ()))
        s_scr[...] = lax.dot_general(q, k1, dn, preferred_element_type=F32)
        _extract_topk(s_scr, v1_scr, i1_scr, NSK, TK)
        s_scr[...] = lax.dot_general(q, k2, dn, preferred_element_type=F32)
        _extract_topk(s_scr, v2_scr, i2_scr, NSK, TK)

        va = v1_scr[...]
        vb = v2_scr[...]
        iota_k = lax.broadcasted_iota(I32, (TOKB, TK), 1)
        va_masked = jnp.where(iota_k < 4, -jnp.inf, va)
        for i in range(4):
            c_scr[:, i * TK:(i + 1) * TK] = v1_scr[:, i:i + 1] + vb
        for j in range(8):
            c_scr[:, 128 + j * TK:128 + (j + 1) * TK] = (
                va_masked + v2_scr[:, j:j + 1])

        iota_cw = lax.broadcasted_iota(I32, (TOKB, CW), 1)
        i1 = i1_scr[...]
        i2 = i2_scr[...]

        def body(m, _):
            c = c_scr[...]
            mx = jnp.max(c, axis=1, keepdims=True)
            am = jnp.min(jnp.where(c == mx, iota_cw, CW), axis=1, keepdims=True)
            c_scr[...] = jnp.where(iota_cw == am, -jnp.inf, c)
            lb = am - 128
            in_a = am < 128
            row = jnp.where(in_a, am // TK, lb % TK)
            col = jnp.where(in_a, am % TK, lb // TK)
            idx1 = jnp.sum(jnp.where(iota_k == row, i1, 0), axis=1, keepdims=True)
            idx2 = jnp.sum(jnp.where(iota_k == col, i2, 0), axis=1, keepdims=True)
            fidx_ref[...] = jnp.where(iota_k == m, idx1 * NSK + idx2, fidx_ref[...])
            w_ref[...] = jnp.where(iota_k == m, mx, w_ref[...])
            return 0

        lax.fori_loop(0, TK, body, 0)

        cs = w_ref[...]
        e = jnp.exp(cs - jnp.max(cs, axis=1, keepdims=True))
        w_ref[...] = e / jnp.sum(e, axis=1, keepdims=True)


def _mlp_topk(x2, wq1, bq1, wq2, bq2, sk1, sk2, interpret=False):
    n = x2.shape[0]
    ntb = n // TOKB
    grid = (ntb, NHB)
    return pl.pallas_call(
        _mlp_topk_body,
        grid=grid,
        in_specs=[
            pl.BlockSpec((TOKB, D), lambda tb, hb: (tb, 0)),
            pl.BlockSpec((D, HIDB), lambda tb, hb: (0, hb)),
            pl.BlockSpec((1, HIDB), lambda tb, hb: (0, hb)),
            pl.BlockSpec((HIDB, SKD), lambda tb, hb: (hb, 0)),
            pl.BlockSpec((1, SKD), lambda tb, hb: (0, 0)),
            pl.BlockSpec((NSK, SKD), lambda tb, hb: (0, 0)),
            pl.BlockSpec((NSK, SKD), lambda tb, hb: (0, 0)),
        ],
        out_specs=[
            pl.BlockSpec((TOKB, TK), lambda tb, hb: (tb, 0)),
            pl.BlockSpec((TOKB, TK), lambda tb, hb: (tb, 0)),
        ],
        out_shape=[
            jax.ShapeDtypeStruct((n, TK), I32),
            jax.ShapeDtypeStruct((n, TK), F32),
        ],
        scratch_shapes=[
            pltpu.VMEM((TOKB, SKD), F32),
            pltpu.VMEM((TOKB, NSK), F32),
            pltpu.VMEM((TOKB, CW), F32),
            pltpu.VMEM((TOKB, TK), F32),
            pltpu.VMEM((TOKB, TK), I32),
            pltpu.VMEM((TOKB, TK), F32),
            pltpu.VMEM((TOKB, TK), I32),
        ],
        compiler_params=pltpu.CompilerParams(
            dimension_semantics=("parallel", "arbitrary")),
        interpret=interpret,
    )(x2, wq1, bq1, wq2, bq2, sk1, sk2)


def _out_mlp_body(x_ref, y_ref, w1_ref, w2_ref, o_ref):
    m = _silu(jnp.dot(x_ref[...], w1_ref[...], preferred_element_type=F32))
    o_ref[...] = jnp.dot(y_ref[...] * m, w2_ref[...], preferred_element_type=F32)


def _out_mlp(x2, y, w1, w2, interpret=False):
    n = x2.shape[0]
    return pl.pallas_call(
        _out_mlp_body,
        grid=(n // TOKB,),
        in_specs=[
            pl.BlockSpec((TOKB, D), lambda tb: (tb, 0)),
            pl.BlockSpec((TOKB, VD), lambda tb: (tb, 0)),
            pl.BlockSpec((D, VD), lambda tb: (0, 0)),
            pl.BlockSpec((VD, D), lambda tb: (0, 0)),
        ],
        out_specs=pl.BlockSpec((TOKB, D), lambda tb: (tb, 0)),
        out_shape=jax.ShapeDtypeStruct((n, D), F32),
        compiler_params=pltpu.CompilerParams(
            dimension_semantics=("parallel",)),
        interpret=interpret,
    )(x2, y, w1, w2)


def _sc_weighted_gather(values, fidx, w16):
    n = fidx.shape[0]
    tpw = n // NW
    mesh = plsc.VectorSubcoreMesh(core_axis_name="c", subcore_axis_name="s")

    @functools.partial(
        pl.kernel, mesh=mesh,
        out_type=jax.ShapeDtypeStruct((n, VD), F32),
        scratch_types=[
            pltpu.VMEM((tpw, TK), I32),
            pltpu.VMEM((TK, VD), F32),
            pltpu.VMEM((TK, VD), F32),
            pltpu.VMEM((TK, 16), F32),
            pltpu.VMEM((TK, 16), F32),
            pltpu.VMEM((VD,), F32),
            pltpu.VMEM((VD,), F32),
            pltpu.SemaphoreType.DMA,
            pltpu.SemaphoreType.DMA,
            pltpu.SemaphoreType.DMA,
            pltpu.SemaphoreType.DMA,
            pltpu.SemaphoreType.DMA,
            pltpu.SemaphoreType.DMA,
        ],
    )
    def k(values_hbm, fidx_hbm, w_hbm, out_hbm, idx_v,
          rows_a, rows_b, w_a, w_b, y_a, y_b,
          sga, sgb, swa, swb, sya, syb):
        wid = lax.axis_index("s") * NC + lax.axis_index("c")
        base = wid * tpw
        pltpu.sync_copy(fidx_hbm.at[pl.ds(base, tpw)], idx_v)

        def start_fetch(t, rows_v, w_v, sg, sw):
            pltpu.make_async_copy(values_hbm.at[idx_v.at[t]], rows_v, sg).start()
            pltpu.make_async_copy(w_hbm.at[base + t], w_v, sw).start()

        def wait_fetch(t, rows_v, w_v, sg, sw):
            pltpu.make_async_copy(values_hbm.at[idx_v.at[t]], rows_v, sg).wait()
            pltpu.make_async_copy(w_hbm.at[base + t], w_v, sw).wait()

        def compute(rows_v, w_v, y_v):
            for dblk in range(4):
                def kbody(kk, accs):
                    wk = w_v[kk, :]
                    return tuple(
                        accs[d] + rows_v[kk, pl.ds(dblk * 256 + d * 16, 16)] * wk
                        for d in range(16))
                accs = lax.fori_loop(
                    0, TK, kbody,
                    tuple(jnp.zeros((16,), F32) for _ in range(16)))
                for d in range(16):
                    y_v[pl.ds(dblk * 256 + d * 16, 16)] = accs[d]

        start_fetch(0, rows_a, w_a, sga, swa)
        start_fetch(1, rows_b, w_b, sgb, swb)

        def pair(g, _):
            for (t, rows_v, w_v, y_v, sg, sw, sy) in (
                    (2 * g, rows_a, w_a, y_a, sga, swa, sya),
                    (2 * g + 1, rows_b, w_b, y_b, sgb, swb, syb)):
                wait_fetch(t, rows_v, w_v, sg, sw)

                @pl.when(g > 0)
                def _drain():
                    pltpu.make_async_copy(y_v, out_hbm.at[base + t], sy).wait()

                compute(rows_v, w_v, y_v)
                pltpu.make_async_copy(y_v, out_hbm.at[base + t], sy).start()

                @pl.when(t + 2 < tpw)
                def _next():
                    start_fetch(t + 2, rows_v, w_v, sg, sw)
            return 0

        lax.fori_loop(0, tpw // 2, pair, 0)
        pltpu.make_async_copy(y_a, out_hbm.at[base], sya).wait()
        pltpu.make_async_copy(y_b, out_hbm.at[base], syb).wait()

    return k(values, fidx, w16)


def kernel(x, Wq1, bq1, Wq2, bq2, subkey_one, subkey_two, values, W1, W2):
    b, s, d = x.shape
    n = b * s
    x2 = x.reshape(n, d)
    fidx, w = _mlp_topk(x2, Wq1, bq1.reshape(1, -1), Wq2, bq2.reshape(1, -1),
                        subkey_one, subkey_two)
    w16 = jnp.broadcast_to(w[:, :, None], (n, TK, 16))
    y = _sc_weighted_gather(values, fidx, w16)
    out2 = _out_mlp(x2, y, W1, W2)
    return out2.reshape(b, s, d)


# speedup vs baseline: 4.5866x; 1.2697x over previous
import functools

import jax
import jax.numpy as jnp
from jax import lax
from jax.experimental import pallas as pl
from jax.experimental.pallas import tpu as pltpu
from jax.experimental.pallas import tpu_sc as plsc

F32 = jnp.float32
I32 = jnp.int32

D = 1024
HID = 4096
SKD = 256
NSK = 256
VD = 1024
TK = 32

TOKB = 512
HIDB = 1024
NHB = HID // HIDB
CW = 128 + 8 * TK

NC, NS = 2, 16
NW = NC * NS


def _rms(v, axis=-1):
    return v * lax.rsqrt(jnp.mean(v * v, axis=axis, keepdims=True) + 1e-6)


def _silu(v):
    return v * (1.0 / (1.0 + jnp.exp(-v)))


def _extract_topk(s_scr, vals_ref, idx_ref, width, k):
    iota_w = lax.broadcasted_iota(I32, s_scr.shape, 1)
    iota_k = lax.broadcasted_iota(I32, idx_ref.shape, 1)

    def body(m, _):
        s = s_scr[...]
        mx = jnp.max(s, axis=1, keepdims=True)
        am = jnp.min(jnp.where(s == mx, iota_w, width), axis=1, keepdims=True)
        vals_ref[...] = jnp.where(iota_k == m, mx, vals_ref[...])
        idx_ref[...] = jnp.where(iota_k == m, am, idx_ref[...])
        s_scr[...] = jnp.where(iota_w == am, -jnp.inf, s)
        return 0

    lax.fori_loop(0, k, body, 0)


def _mlp_topk_body(x_ref, wq1_ref, bq1_ref, wq2_ref, bq2_ref, sk1_ref, sk2_ref,
                   fidx_ref, w_ref,
                   qacc, s_scr, c_scr, v1_scr, i1_scr, v2_scr, i2_scr):
    hb = pl.program_id(1)

    @pl.when(hb == 0)
    def _init():
        qacc[...] = jnp.zeros_like(qacc)

    h = _silu(jnp.dot(x_ref[...], wq1_ref[...], preferred_element_type=F32)
              + bq1_ref[...])
    qacc[...] += jnp.dot(h, wq2_ref[...], preferred_element_type=F32)

    @pl.when(hb == NHB - 1)
    def _finish():
        q = _rms(qacc[...] + bq2_ref[...])
        k1 = _rms(sk1_ref[...])
        k2 = _rms(sk2_ref[...])
        dn = (((1,), (1,)), ((), ()))
        s_scr[...] = lax.dot_general(q, k1, dn, preferred_element_type=F32)
        _extract_topk(s_scr, v1_scr, i1_scr, NSK, TK)
        s_scr[...] = lax.dot_general(q, k2, dn, preferred_element_type=F32)
        _extract_topk(s_scr, v2_scr, i2_scr, NSK, TK)

        va = v1_scr[...]
        vb = v2_scr[...]
        iota_k = lax.broadcasted_iota(I32, (TOKB, TK), 1)
        va_masked = jnp.where(iota_k < 4, -jnp.inf, va)
        for i in range(4):
            c_scr[:, i * TK:(i + 1) * TK] = v1_scr[:, i:i + 1] + vb
        for j in range(8):
            c_scr[:, 128 + j * TK:128 + (j + 1) * TK] = (
                va_masked + v2_scr[:, j:j + 1])

        iota_cw = lax.broadcasted_iota(I32, (TOKB, CW), 1)
        i1 = i1_scr[...]
        i2 = i2_scr[...]

        def body(m, _):
            c = c_scr[...]
            mx = jnp.max(c, axis=1, keepdims=True)
            am = jnp.min(jnp.where(c == mx, iota_cw, CW), axis=1, keepdims=True)
            c_scr[...] = jnp.where(iota_cw == am, -jnp.inf, c)
            lb = am - 128
            in_a = am < 128
            row = jnp.where(in_a, am // TK, lb % TK)
            col = jnp.where(in_a, am % TK, lb // TK)
            idx1 = jnp.sum(jnp.where(iota_k == row, i1, 0), axis=1, keepdims=True)
            idx2 = jnp.sum(jnp.where(iota_k == col, i2, 0), axis=1, keepdims=True)
            fidx_ref[...] = jnp.where(iota_k == m, idx1 * NSK + idx2, fidx_ref[...])
            w_ref[...] = jnp.where(iota_k == m, mx, w_ref[...])
            return 0

        lax.fori_loop(0, TK, body, 0)

        cs = w_ref[...]
        e = jnp.exp(cs - jnp.max(cs, axis=1, keepdims=True))
        w_ref[...] = e / jnp.sum(e, axis=1, keepdims=True)


def _mlp_topk(x2, wq1, bq1, wq2, bq2, sk1, sk2, interpret=False):
    n = x2.shape[0]
    ntb = n // TOKB
    grid = (ntb, NHB)
    return pl.pallas_call(
        _mlp_topk_body,
        grid=grid,
        in_specs=[
            pl.BlockSpec((TOKB, D), lambda tb, hb: (tb, 0)),
            pl.BlockSpec((D, HIDB), lambda tb, hb: (0, hb)),
            pl.BlockSpec((1, HIDB), lambda tb, hb: (0, hb)),
            pl.BlockSpec((HIDB, SKD), lambda tb, hb: (hb, 0)),
            pl.BlockSpec((1, SKD), lambda tb, hb: (0, 0)),
            pl.BlockSpec((NSK, SKD), lambda tb, hb: (0, 0)),
            pl.BlockSpec((NSK, SKD), lambda tb, hb: (0, 0)),
        ],
        out_specs=[
            pl.BlockSpec((TOKB, TK), lambda tb, hb: (tb, 0)),
            pl.BlockSpec((TOKB, TK), lambda tb, hb: (tb, 0)),
        ],
        out_shape=[
            jax.ShapeDtypeStruct((n, TK), I32),
            jax.ShapeDtypeStruct((n, TK), F32),
        ],
        scratch_shapes=[
            pltpu.VMEM((TOKB, SKD), F32),
            pltpu.VMEM((TOKB, NSK), F32),
            pltpu.VMEM((TOKB, CW), F32),
            pltpu.VMEM((TOKB, TK), F32),
            pltpu.VMEM((TOKB, TK), I32),
            pltpu.VMEM((TOKB, TK), F32),
            pltpu.VMEM((TOKB, TK), I32),
        ],
        compiler_params=pltpu.CompilerParams(
            dimension_semantics=("parallel", "arbitrary")),
        interpret=interpret,
    )(x2, wq1, bq1, wq2, bq2, sk1, sk2)


def _out_mlp_body(x_ref, y_ref, w1_ref, w2_ref, o_ref):
    m = _silu(jnp.dot(x_ref[...], w1_ref[...], preferred_element_type=F32))
    o_ref[...] = jnp.dot(y_ref[...] * m, w2_ref[...], preferred_element_type=F32)


def _out_mlp(x2, y, w1, w2, interpret=False):
    n = x2.shape[0]
    return pl.pallas_call(
        _out_mlp_body,
        grid=(n // TOKB,),
        in_specs=[
            pl.BlockSpec((TOKB, D), lambda tb: (tb, 0)),
            pl.BlockSpec((TOKB, VD), lambda tb: (tb, 0)),
            pl.BlockSpec((D, VD), lambda tb: (0, 0)),
            pl.BlockSpec((VD, D), lambda tb: (0, 0)),
        ],
        out_specs=pl.BlockSpec((TOKB, D), lambda tb: (tb, 0)),
        out_shape=jax.ShapeDtypeStruct((n, D), F32),
        compiler_params=pltpu.CompilerParams(
            dimension_semantics=("parallel",)),
        interpret=interpret,
    )(x2, y, w1, w2)


def _sc_weighted_gather(values, fidx, w16):
    n = fidx.shape[0]
    tpw = n // NW
    mesh = plsc.VectorSubcoreMesh(core_axis_name="c", subcore_axis_name="s")

    @functools.partial(
        pl.kernel, mesh=mesh,
        out_type=jax.ShapeDtypeStruct((n, VD), F32),
        scratch_types=[
            pltpu.VMEM((tpw, TK), I32),
            pltpu.VMEM((TK, VD), F32),
            pltpu.VMEM((TK, VD), F32),
            pltpu.VMEM((TK, 16), F32),
            pltpu.VMEM((TK, 16), F32),
            pltpu.VMEM((VD,), F32),
            pltpu.VMEM((VD,), F32),
            pltpu.SemaphoreType.DMA,
            pltpu.SemaphoreType.DMA,
            pltpu.SemaphoreType.DMA,
            pltpu.SemaphoreType.DMA,
            pltpu.SemaphoreType.DMA,
            pltpu.SemaphoreType.DMA,
        ],
    )
    def k(values_hbm, fidx_hbm, w_hbm, out_hbm, idx_v,
          rows_a, rows_b, w_a, w_b, y_a, y_b,
          sga, sgb, swa, swb, sya, syb):
        wid = lax.axis_index("s") * NC + lax.axis_index("c")
        base = wid * tpw
        pltpu.sync_copy(fidx_hbm.at[pl.ds(base, tpw)], idx_v)

        def start_fetch(t, rows_v, w_v, sg, sw):
            pltpu.make_async_copy(values_hbm.at[idx_v.at[t]], rows_v, sg).start()
            pltpu.make_async_copy(w_hbm.at[base + t], w_v, sw).start()

        def wait_fetch(t, rows_v, w_v, sg, sw):
            pltpu.make_async_copy(values_hbm.at[idx_v.at[t]], rows_v, sg).wait()
            pltpu.make_async_copy(w_hbm.at[base + t], w_v, sw).wait()

        def compute(rows_v, w_v, y_v):
            for dblk in range(4):
                def kbody(kk, accs):
                    wk = w_v[kk, :]
                    return tuple(
                        accs[d] + rows_v[kk, pl.ds(dblk * 256 + d * 16, 16)] * wk
                        for d in range(16))
                accs = lax.fori_loop(
                    0, TK, kbody,
                    tuple(jnp.zeros((16,), F32) for _ in range(16)))
                for d in range(16):
                    y_v[pl.ds(dblk * 256 + d * 16, 16)] = accs[d]

        start_fetch(0, rows_a, w_a, sga, swa)
        start_fetch(1, rows_b, w_b, sgb, swb)

        def pair(g, _):
            for (t, rows_v, w_v, y_v, sg, sw, sy) in (
                    (2 * g, rows_a, w_a, y_a, sga, swa, sya),
                    (2 * g + 1, rows_b, w_b, y_b, sgb, swb, syb)):
                wait_fetch(t, rows_v, w_v, sg, sw)

                @pl.when(g > 0)
                def _drain():
                    pltpu.make_async_copy(y_v, out_hbm.at[base + t], sy).wait()

                compute(rows_v, w_v, y_v)
                pltpu.make_async_copy(y_v, out_hbm.at[base + t], sy).start()

                @pl.when(t + 2 < tpw)
                def _next():
                    start_fetch(t + 2, rows_v, w_v, sg, sw)
            return 0

        lax.fori_loop(0, tpw // 2, pair, 0)
        pltpu.make_async_copy(y_a, out_hbm.at[base], sya).wait()
        pltpu.make_async_copy(y_b, out_hbm.at[base], syb).wait()

    return k(values, fidx, w16)


def kernel(x, Wq1, bq1, Wq2, bq2, subkey_one, subkey_two, values, W1, W2):
    b, s, d = x.shape
    n = b * s
    x2 = x.reshape(n, d)
    fidx, w = _mlp_topk(x2, Wq1, bq1.reshape(1, -1), Wq2, bq2.reshape(1, -1),
                        subkey_one, subkey_two)
    w16 = jnp.broadcast_to(w[:, :, None], (n, TK, 16))
    y = _sc_weighted_gather(values, fidx, w16)
    out2 = _out_mlp(x2, y, W1, W2)
    return out2.reshape(b, s, d)


# speedup vs baseline: 4.9811x; 1.0860x over previous
import functools

import jax
import jax.numpy as jnp
from jax import lax
from jax.experimental import pallas as pl
from jax.experimental.pallas import tpu as pltpu
from jax.experimental.pallas import tpu_sc as plsc

F32 = jnp.float32
I32 = jnp.int32

D = 1024
HID = 4096
SKD = 256
NSK = 256
VD = 1024
TK = 32

TOKB = 1024
HIDB = 1024
NHB = HID // HIDB
CW = 128 + 8 * TK

NC, NS = 2, 16
NW = NC * NS


def _rms(v, axis=-1):
    return v * lax.rsqrt(jnp.mean(v * v, axis=axis, keepdims=True) + 1e-6)


def _silu(v):
    return v * (1.0 / (1.0 + jnp.exp(-v)))


def _extract_topk(s_scr, vals_ref, idx_ref, width, k):
    iota_w = lax.broadcasted_iota(I32, s_scr.shape, 1)
    iota_k = lax.broadcasted_iota(I32, idx_ref.shape, 1)

    def body(m, _):
        s = s_scr[...]
        mx = jnp.max(s, axis=1, keepdims=True)
        am = jnp.min(jnp.where(s == mx, iota_w, width), axis=1, keepdims=True)
        vals_ref[...] = jnp.where(iota_k == m, mx, vals_ref[...])
        idx_ref[...] = jnp.where(iota_k == m, am, idx_ref[...])
        s_scr[...] = jnp.where(iota_w == am, -jnp.inf, s)
        return 0

    lax.fori_loop(0, k, body, 0)


def _mlp_topk_body(x_ref, wq1_ref, bq1_ref, wq2_ref, bq2_ref, sk1_ref, sk2_ref,
                   fidx_ref, w_ref,
                   qacc, s_scr, c_scr, v1_scr, i1_scr, v2_scr, i2_scr):
    hb = pl.program_id(1)

    @pl.when(hb == 0)
    def _init():
        qacc[...] = jnp.zeros_like(qacc)

    h = _silu(jnp.dot(x_ref[...], wq1_ref[...], preferred_element_type=F32)
              + bq1_ref[...])
    qacc[...] += jnp.dot(h, wq2_ref[...], preferred_element_type=F32)

    @pl.when(hb == NHB - 1)
    def _finish():
        q = _rms(qacc[...] + bq2_ref[...])
        k1 = _rms(sk1_ref[...])
        k2 = _rms(sk2_ref[...])
        dn = (((1,), (1,)), ((), ()))
        s_scr[...] = lax.dot_general(q, k1, dn, preferred_element_type=F32)
        _extract_topk(s_scr, v1_scr, i1_scr, NSK, TK)
        s_scr[...] = lax.dot_general(q, k2, dn, preferred_element_type=F32)
        _extract_topk(s_scr, v2_scr, i2_scr, NSK, TK)

        va = v1_scr[...]
        vb = v2_scr[...]
        iota_k = lax.broadcasted_iota(I32, (TOKB, TK), 1)
        va_masked = jnp.where(iota_k < 4, -jnp.inf, va)
        for i in range(4):
            c_scr[:, i * TK:(i + 1) * TK] = v1_scr[:, i:i + 1] + vb
        for j in range(8):
            c_scr[:, 128 + j * TK:128 + (j + 1) * TK] = (
                va_masked + v2_scr[:, j:j + 1])

        iota_cw = lax.broadcasted_iota(I32, (TOKB, CW), 1)
        i1 = i1_scr[...]
        i2 = i2_scr[...]

        def body(m, _):
            c = c_scr[...]
            mx = jnp.max(c, axis=1, keepdims=True)
            am = jnp.min(jnp.where(c == mx, iota_cw, CW), axis=1, keepdims=True)
            c_scr[...] = jnp.where(iota_cw == am, -jnp.inf, c)
            lb = am - 128
            in_a = am < 128
            row = jnp.where(in_a, am // TK, lb % TK)
            col = jnp.where(in_a, am % TK, lb // TK)
            idx1 = jnp.sum(jnp.where(iota_k == row, i1, 0), axis=1, keepdims=True)
            idx2 = jnp.sum(jnp.where(iota_k == col, i2, 0), axis=1, keepdims=True)
            fidx_ref[...] = jnp.where(iota_k == m, idx1 * NSK + idx2, fidx_ref[...])
            w_ref[...] = jnp.where(iota_k == m, mx, w_ref[...])
            return 0

        lax.fori_loop(0, TK, body, 0)

        cs = w_ref[...]
        e = jnp.exp(cs - jnp.max(cs, axis=1, keepdims=True))
        w_ref[...] = e / jnp.sum(e, axis=1, keepdims=True)


def _mlp_topk(x2, wq1, bq1, wq2, bq2, sk1, sk2, interpret=False):
    n = x2.shape[0]
    ntb = n // TOKB
    grid = (ntb, NHB)
    return pl.pallas_call(
        _mlp_topk_body,
        grid=grid,
        in_specs=[
            pl.BlockSpec((TOKB, D), lambda tb, hb: (tb, 0)),
            pl.BlockSpec((D, HIDB), lambda tb, hb: (0, hb)),
            pl.BlockSpec((1, HIDB), lambda tb, hb: (0, hb)),
            pl.BlockSpec((HIDB, SKD), lambda tb, hb: (hb, 0)),
            pl.BlockSpec((1, SKD), lambda tb, hb: (0, 0)),
            pl.BlockSpec((NSK, SKD), lambda tb, hb: (0, 0)),
            pl.BlockSpec((NSK, SKD), lambda tb, hb: (0, 0)),
        ],
        out_specs=[
            pl.BlockSpec((TOKB, TK), lambda tb, hb: (tb, 0)),
            pl.BlockSpec((TOKB, TK), lambda tb, hb: (tb, 0)),
        ],
        out_shape=[
            jax.ShapeDtypeStruct((n, TK), I32),
            jax.ShapeDtypeStruct((n, TK), F32),
        ],
        scratch_shapes=[
            pltpu.VMEM((TOKB, SKD), F32),
            pltpu.VMEM((TOKB, NSK), F32),
            pltpu.VMEM((TOKB, CW), F32),
            pltpu.VMEM((TOKB, TK), F32),
            pltpu.VMEM((TOKB, TK), I32),
            pltpu.VMEM((TOKB, TK), F32),
            pltpu.VMEM((TOKB, TK), I32),
        ],
        compiler_params=pltpu.CompilerParams(
            dimension_semantics=("parallel", "arbitrary")),
        interpret=interpret,
    )(x2, wq1, bq1, wq2, bq2, sk1, sk2)


def _out_mlp_body(x_ref, y_ref, w1_ref, w2_ref, o_ref):
    m = _silu(jnp.dot(x_ref[...], w1_ref[...], preferred_element_type=F32))
    o_ref[...] = jnp.dot(y_ref[...] * m, w2_ref[...], preferred_element_type=F32)


def _out_mlp(x2, y, w1, w2, interpret=False):
    n = x2.shape[0]
    return pl.pallas_call(
        _out_mlp_body,
        grid=(n // TOKB,),
        in_specs=[
            pl.BlockSpec((TOKB, D), lambda tb: (tb, 0)),
            pl.BlockSpec((TOKB, VD), lambda tb: (tb, 0)),
            pl.BlockSpec((D, VD), lambda tb: (0, 0)),
            pl.BlockSpec((VD, D), lambda tb: (0, 0)),
        ],
        out_specs=pl.BlockSpec((TOKB, D), lambda tb: (tb, 0)),
        out_shape=jax.ShapeDtypeStruct((n, D), F32),
        compiler_params=pltpu.CompilerParams(
            dimension_semantics=("parallel",)),
        interpret=interpret,
    )(x2, y, w1, w2)


def _sc_weighted_gather(values, fidx, w16):
    n = fidx.shape[0]
    tpw = n // NW
    mesh = plsc.VectorSubcoreMesh(core_axis_name="c", subcore_axis_name="s")

    @functools.partial(
        pl.kernel, mesh=mesh,
        out_type=jax.ShapeDtypeStruct((n, VD), F32),
        scratch_types=[
            pltpu.VMEM((tpw, TK), I32),
            pltpu.VMEM((TK, VD), F32),
            pltpu.VMEM((TK, VD), F32),
            pltpu.VMEM((TK, 16), F32),
            pltpu.VMEM((TK, 16), F32),
            pltpu.VMEM((VD,), F32),
            pltpu.VMEM((VD,), F32),
            pltpu.SemaphoreType.DMA,
            pltpu.SemaphoreType.DMA,
            pltpu.SemaphoreType.DMA,
            pltpu.SemaphoreType.DMA,
            pltpu.SemaphoreType.DMA,
            pltpu.SemaphoreType.DMA,
        ],
    )
    def k(values_hbm, fidx_hbm, w_hbm, out_hbm, idx_v,
          rows_a, rows_b, w_a, w_b, y_a, y_b,
          sga, sgb, swa, swb, sya, syb):
        wid = lax.axis_index("s") * NC + lax.axis_index("c")
        base = wid * tpw
        pltpu.sync_copy(fidx_hbm.at[pl.ds(base, tpw)], idx_v)

        def start_fetch(t, rows_v, w_v, sg, sw):
            pltpu.make_async_copy(values_hbm.at[idx_v.at[t]], rows_v, sg).start()
            pltpu.make_async_copy(w_hbm.at[base + t], w_v, sw).start()

        def wait_fetch(t, rows_v, w_v, sg, sw):
            pltpu.make_async_copy(values_hbm.at[idx_v.at[t]], rows_v, sg).wait()
            pltpu.make_async_copy(w_hbm.at[base + t], w_v, sw).wait()

        def compute(rows_v, w_v, y_v):
            for dblk in range(4):
                def kbody(kk, accs):
                    wk = w_v[kk, :]
                    return tuple(
                        accs[d] + rows_v[kk, pl.ds(dblk * 256 + d * 16, 16)] * wk
                        for d in range(16))
                accs = lax.fori_loop(
                    0, TK, kbody,
                    tuple(jnp.zeros((16,), F32) for _ in range(16)))
                for d in range(16):
                    y_v[pl.ds(dblk * 256 + d * 16, 16)] = accs[d]

        start_fetch(0, rows_a, w_a, sga, swa)
        start_fetch(1, rows_b, w_b, sgb, swb)

        def pair(g, _):
            for (t, rows_v, w_v, y_v, sg, sw, sy) in (
                    (2 * g, rows_a, w_a, y_a, sga, swa, sya),
                    (2 * g + 1, rows_b, w_b, y_b, sgb, swb, syb)):
                wait_fetch(t, rows_v, w_v, sg, sw)

                @pl.when(g > 0)
                def _drain():
                    pltpu.make_async_copy(y_v, out_hbm.at[base + t], sy).wait()

                compute(rows_v, w_v, y_v)
                pltpu.make_async_copy(y_v, out_hbm.at[base + t], sy).start()

                @pl.when(t + 2 < tpw)
                def _next():
                    start_fetch(t + 2, rows_v, w_v, sg, sw)
            return 0

        lax.fori_loop(0, tpw // 2, pair, 0)
        pltpu.make_async_copy(y_a, out_hbm.at[base], sya).wait()
        pltpu.make_async_copy(y_b, out_hbm.at[base], syb).wait()

    return k(values, fidx, w16)


def kernel(x, Wq1, bq1, Wq2, bq2, subkey_one, subkey_two, values, W1, W2):
    b, s, d = x.shape
    n = b * s
    x2 = x.reshape(n, d)
    fidx, w = _mlp_topk(x2, Wq1, bq1.reshape(1, -1), Wq2, bq2.reshape(1, -1),
                        subkey_one, subkey_two)
    w16 = jnp.broadcast_to(w[:, :, None], (n, TK, 16))
    y = _sc_weighted_gather(values, fidx, w16)
    out2 = _out_mlp(x2, y, W1, W2)
    return out2.reshape(b, s, d)


# speedup vs baseline: 5.1072x; 1.0253x over previous
import functools

import jax
import jax.numpy as jnp
from jax import lax
from jax.experimental import pallas as pl
from jax.experimental.pallas import tpu as pltpu
from jax.experimental.pallas import tpu_sc as plsc

F32 = jnp.float32
I32 = jnp.int32

D = 1024
HID = 4096
SKD = 256
NSK = 256
VD = 1024
TK = 32

TOKB = 1024
HIDB = 1024
NHB = HID // HIDB
CW = 128 + 8 * TK

NC, NS = 2, 16
NW = NC * NS


def _rms(v, axis=-1):
    return v * lax.rsqrt(jnp.mean(v * v, axis=axis, keepdims=True) + 1e-6)


def _silu(v):
    return v * (1.0 / (1.0 + jnp.exp(-v)))


def _extract_topk(s_scr, vals_ref, idx_ref, width, k):
    iota_w = lax.broadcasted_iota(I32, s_scr.shape, 1)
    iota_k = lax.broadcasted_iota(I32, idx_ref.shape, 1)

    def body(m, _):
        s = s_scr[...]
        mx = jnp.max(s, axis=1, keepdims=True)
        am = jnp.min(jnp.where(s == mx, iota_w, width), axis=1, keepdims=True)
        vals_ref[...] = jnp.where(iota_k == m, mx, vals_ref[...])
        idx_ref[...] = jnp.where(iota_k == m, am, idx_ref[...])
        s_scr[...] = jnp.where(iota_w == am, -jnp.inf, s)
        return 0

    lax.fori_loop(0, k, body, 0)


def _mlp_topk_body(x_ref, wq1_ref, bq1_ref, wq2_ref, bq2_ref, sk1_ref, sk2_ref,
                   fidx_ref, w_ref,
                   qacc, s_scr, c_scr, v1_scr, i1_scr):
    hb = pl.program_id(1)

    @pl.when(hb == 0)
    def _init():
        qacc[...] = jnp.zeros_like(qacc)

    h = _silu(jnp.dot(x_ref[...], wq1_ref[...], preferred_element_type=F32)
              + bq1_ref[...])
    qacc[...] += jnp.dot(h, wq2_ref[...], preferred_element_type=F32)

    @pl.when(hb == NHB - 1)
    def _finish():
        q = _rms(qacc[...] + bq2_ref[...])
        k1 = _rms(sk1_ref[...])
        k2 = _rms(sk2_ref[...])
        dn = (((1,), (1,)), ((), ()))
        s_scr[0:TOKB, :] = lax.dot_general(q, k1, dn, preferred_element_type=F32)
        s_scr[TOKB:2 * TOKB, :] = lax.dot_general(q, k2, dn,
                                                  preferred_element_type=F32)
        _extract_topk(s_scr, v1_scr, i1_scr, NSK, TK)

        va = v1_scr[0:TOKB, :]
        vb = v1_scr[TOKB:2 * TOKB, :]
        iota_k = lax.broadcasted_iota(I32, (TOKB, TK), 1)
        va_masked = jnp.where(iota_k < 4, -jnp.inf, va)
        for i in range(4):
            c_scr[:, i * TK:(i + 1) * TK] = va[:, i:i + 1] + vb
        for j in range(8):
            c_scr[:, 128 + j * TK:128 + (j + 1) * TK] = va_masked + vb[:, j:j + 1]

        iota_cw = lax.broadcasted_iota(I32, (TOKB, CW), 1)
        i1 = i1_scr[0:TOKB, :]
        i2 = i1_scr[TOKB:2 * TOKB, :]

        def body(m, _):
            c = c_scr[...]
            mx = jnp.max(c, axis=1, keepdims=True)
            am = jnp.min(jnp.where(c == mx, iota_cw, CW), axis=1, keepdims=True)
            c_scr[...] = jnp.where(iota_cw == am, -jnp.inf, c)
            lb = am - 128
            in_a = am < 128
            row = jnp.where(in_a, am // TK, lb % TK)
            col = jnp.where(in_a, am % TK, lb // TK)
            idx1 = jnp.sum(jnp.where(iota_k == row, i1, 0), axis=1, keepdims=True)
            idx2 = jnp.sum(jnp.where(iota_k == col, i2, 0), axis=1, keepdims=True)
            fidx_ref[...] = jnp.where(iota_k == m, idx1 * NSK + idx2, fidx_ref[...])
            w_ref[...] = jnp.where(iota_k == m, mx, w_ref[...])
            return 0

        lax.fori_loop(0, TK, body, 0)

        cs = w_ref[...]
        e = jnp.exp(cs - jnp.max(cs, axis=1, keepdims=True))
        w_ref[...] = e / jnp.sum(e, axis=1, keepdims=True)


def _mlp_topk(x2, wq1, bq1, wq2, bq2, sk1, sk2, interpret=False):
    n = x2.shape[0]
    ntb = n // TOKB
    grid = (ntb, NHB)
    return pl.pallas_call(
        _mlp_topk_body,
        grid=grid,
        in_specs=[
            pl.BlockSpec((TOKB, D), lambda tb, hb: (tb, 0)),
            pl.BlockSpec((D, HIDB), lambda tb, hb: (0, hb)),
            pl.BlockSpec((1, HIDB), lambda tb, hb: (0, hb)),
            pl.BlockSpec((HIDB, SKD), lambda tb, hb: (hb, 0)),
            pl.BlockSpec((1, SKD), lambda tb, hb: (0, 0)),
            pl.BlockSpec((NSK, SKD), lambda tb, hb: (0, 0)),
            pl.BlockSpec((NSK, SKD), lambda tb, hb: (0, 0)),
        ],
        out_specs=[
            pl.BlockSpec((TOKB, TK), lambda tb, hb: (tb, 0)),
            pl.BlockSpec((TOKB, TK), lambda tb, hb: (tb, 0)),
        ],
        out_shape=[
            jax.ShapeDtypeStruct((n, TK), I32),
            jax.ShapeDtypeStruct((n, TK), F32),
        ],
        scratch_shapes=[
            pltpu.VMEM((TOKB, SKD), F32),
            pltpu.VMEM((2 * TOKB, NSK), F32),
            pltpu.VMEM((TOKB, CW), F32),
            pltpu.VMEM((2 * TOKB, TK), F32),
            pltpu.VMEM((2 * TOKB, TK), I32),
        ],
        compiler_params=pltpu.CompilerParams(
            dimension_semantics=("parallel", "arbitrary")),
        interpret=interpret,
    )(x2, wq1, bq1, wq2, bq2, sk1, sk2)


def _out_mlp_body(x_ref, y_ref, w1_ref, w2_ref, o_ref):
    m = _silu(jnp.dot(x_ref[...], w1_ref[...], preferred_element_type=F32))
    o_ref[...] = jnp.dot(y_ref[...] * m, w2_ref[...], preferred_element_type=F32)


def _out_mlp(x2, y, w1, w2, interpret=False):
    n = x2.shape[0]
    return pl.pallas_call(
        _out_mlp_body,
        grid=(n // TOKB,),
        in_specs=[
            pl.BlockSpec((TOKB, D), lambda tb: (tb, 0)),
            pl.BlockSpec((TOKB, VD), lambda tb: (tb, 0)),
            pl.BlockSpec((D, VD), lambda tb: (0, 0)),
            pl.BlockSpec((VD, D), lambda tb: (0, 0)),
        ],
        out_specs=pl.BlockSpec((TOKB, D), lambda tb: (tb, 0)),
        out_shape=jax.ShapeDtypeStruct((n, D), F32),
        compiler_params=pltpu.CompilerParams(
            dimension_semantics=("parallel",)),
        interpret=interpret,
    )(x2, y, w1, w2)


def _sc_weighted_gather(values, fidx, w16):
    n = fidx.shape[0]
    tpw = n // NW
    mesh = plsc.VectorSubcoreMesh(core_axis_name="c", subcore_axis_name="s")

    @functools.partial(
        pl.kernel, mesh=mesh,
        out_type=jax.ShapeDtypeStruct((n, VD), F32),
        scratch_types=[
            pltpu.VMEM((tpw, TK), I32),
            pltpu.VMEM((TK, VD), F32),
            pltpu.VMEM((TK, VD), F32),
            pltpu.VMEM((TK, 16), F32),
            pltpu.VMEM((TK, 16), F32),
            pltpu.VMEM((VD,), F32),
            pltpu.VMEM((VD,), F32),
            pltpu.SemaphoreType.DMA,
            pltpu.SemaphoreType.DMA,
            pltpu.SemaphoreType.DMA,
            pltpu.SemaphoreType.DMA,
            pltpu.SemaphoreType.DMA,
            pltpu.SemaphoreType.DMA,
        ],
    )
    def k(values_hbm, fidx_hbm, w_hbm, out_hbm, idx_v,
          rows_a, rows_b, w_a, w_b, y_a, y_b,
          sga, sgb, swa, swb, sya, syb):
        wid = lax.axis_index("s") * NC + lax.axis_index("c")
        base = wid * tpw
        pltpu.sync_copy(fidx_hbm.at[pl.ds(base, tpw)], idx_v)

        def start_fetch(t, rows_v, w_v, sg, sw):
            pltpu.make_async_copy(values_hbm.at[idx_v.at[t]], rows_v, sg).start()
            pltpu.make_async_copy(w_hbm.at[base + t], w_v, sw).start()

        def wait_fetch(t, rows_v, w_v, sg, sw):
            pltpu.make_async_copy(values_hbm.at[idx_v.at[t]], rows_v, sg).wait()
            pltpu.make_async_copy(w_hbm.at[base + t], w_v, sw).wait()

        def compute(rows_v, w_v, y_v):
            for dblk in range(4):
                def kbody(kk, accs):
                    wk = w_v[kk, :]
                    return tuple(
                        accs[d] + rows_v[kk, pl.ds(dblk * 256 + d * 16, 16)] * wk
                        for d in range(16))
                accs = lax.fori_loop(
                    0, TK, kbody,
                    tuple(jnp.zeros((16,), F32) for _ in range(16)))
                for d in range(16):
                    y_v[pl.ds(dblk * 256 + d * 16, 16)] = accs[d]

        start_fetch(0, rows_a, w_a, sga, swa)
        start_fetch(1, rows_b, w_b, sgb, swb)

        def pair(g, _):
            for (t, rows_v, w_v, y_v, sg, sw, sy) in (
                    (2 * g, rows_a, w_a, y_a, sga, swa, sya),
                    (2 * g + 1, rows_b, w_b, y_b, sgb, swb, syb)):
                wait_fetch(t, rows_v, w_v, sg, sw)

                @pl.when(g > 0)
                def _drain():
                    pltpu.make_async_copy(y_v, out_hbm.at[base + t], sy).wait()

                compute(rows_v, w_v, y_v)
                pltpu.make_async_copy(y_v, out_hbm.at[base + t], sy).start()

                @pl.when(t + 2 < tpw)
                def _next():
                    start_fetch(t + 2, rows_v, w_v, sg, sw)
            return 0

        lax.fori_loop(0, tpw // 2, pair, 0)
        pltpu.make_async_copy(y_a, out_hbm.at[base], sya).wait()
        pltpu.make_async_copy(y_b, out_hbm.at[base], syb).wait()

    return k(values, fidx, w16)


def kernel(x, Wq1, bq1, Wq2, bq2, subkey_one, subkey_two, values, W1, W2):
    b, s, d = x.shape
    n = b * s
    x2 = x.reshape(n, d)
    fidx, w = _mlp_topk(x2, Wq1, bq1.reshape(1, -1), Wq2, bq2.reshape(1, -1),
                        subkey_one, subkey_two)
    w16 = jnp.broadcast_to(w[:, :, None], (n, TK, 16))
    y = _sc_weighted_gather(values, fidx, w16)
    out2 = _out_mlp(x2, y, W1, W2)
    return out2.reshape(b, s, d)
